# Initial kernel scaffold; baseline (speedup 1.0000x reference)
import jax
import jax.numpy as jnp
from jax.experimental import pallas as pl


def kernel(idx, embedding):
    raise NotImplementedError("write your pallas kernel here")


# speedup vs baseline: 1.8191x; 1.8191x over previous
import functools

import jax
import jax.numpy as jnp
from jax import lax
from jax.experimental import pallas as pl
from jax.experimental.pallas import tpu as pltpu
from jax.experimental.pallas import tpu_sc as plsc

_VOCAB = 8192
_D = 8192
_B = 4096

_INFO = plsc.get_sparse_core_info()
_NC = _INFO.num_cores
_NS = _INFO.num_subcores
_NW = _NC * _NS
_B_PER_W = _B // _NW
_K = 4
_NCHUNK = _B_PER_W // _K

_MESH = plsc.VectorSubcoreMesh(core_axis_name="c", subcore_axis_name="s")


@functools.partial(
    pl.kernel,
    mesh=_MESH,
    out_type=jax.ShapeDtypeStruct((_B, _D), jnp.float32),
    scratch_types=[
        pltpu.VMEM((_NCHUNK, _K), jnp.int32),
        pltpu.VMEM((2, _K, _D), jnp.float32),
        pltpu.SemaphoreType.DMA,
        pltpu.SemaphoreType.DMA,
        pltpu.SemaphoreType.DMA,
        pltpu.SemaphoreType.DMA,
    ],
)
def _sc_gather(idx_hbm, table_hbm, out_hbm, idx_v, rows_v, g0, g1, o0, o1):
    wid = lax.axis_index("s") * _NC + lax.axis_index("c")
    base = wid * _B_PER_W
    pltpu.sync_copy(idx_hbm.at[wid], idx_v)

    gsem = (g0, g1)
    osem = (o0, o1)

    def gather_start(c):
        buf = c % 2
        return pltpu.async_copy(
            table_hbm.at[idx_v.at[c]], rows_v.at[buf], gsem[buf]
        )

    def out_start(c):
        buf = c % 2
        return pltpu.async_copy(
            rows_v.at[buf], out_hbm.at[pl.ds(base + c * _K, _K)], osem[buf]
        )

    gathers = [None] * _NCHUNK
    outs = [None] * _NCHUNK
    gathers[0] = gather_start(0)
    for c in range(_NCHUNK):
        if c + 1 < _NCHUNK:
            if c - 1 >= 0:
                outs[c - 1].wait()
            gathers[c + 1] = gather_start(c + 1)
        gathers[c].wait()
        outs[c] = out_start(c)
    outs[_NCHUNK - 1].wait()


def kernel(idx, embedding):
    if idx.ndim == 2:
        idx = jnp.squeeze(idx, axis=-1)
    idx3 = idx.astype(jnp.int32).reshape(_NW, _NCHUNK, _K)
    return _sc_gather(idx3, embedding)
